# baseline (speedup 1.0000x reference)
import functools

import jax
import jax.numpy as jnp
from jax import lax
from jax.experimental import pallas as pl
from jax.experimental.pallas import tpu as pltpu
from jax.experimental.pallas import tpu_sc as plsc

B, S, H, E = 4, 2048, 4096, 64
N = B * S
K = 160

TILE = 512
GRID_A = N // TILE

NC, NS = 2, 16
NW = NC * NS
EPW = E // NW
CH = N // 16


def _router_body(x_ref, w_ref, probs_ref, probsT_ref):
    x = x_ref[...]
    w = w_ref[...]
    logits = lax.dot_general(
        x, w, (((1,), (1,)), ((), ())),
        preferred_element_type=jnp.float32,
        precision=lax.Precision.DEFAULT,
    )
    m = jnp.max(logits, axis=1, keepdims=True)
    e = jnp.exp(logits - m)
    probs = e / jnp.sum(e, axis=1, keepdims=True)
    probs_ref[...] = probs
    probsT_ref[...] = lax.bitcast_convert_type(probs.T, jnp.int32)


def _scan16(chunk_loader, nchunks, base_count, k_target, lane, bin_base=0):
    cum = base_count
    best = jnp.int32(-1)
    for c in range(nchunks - 1, -1, -1):
        acc = chunk_loader(c)
        desc = lax.rev(acc, (0,))
        cs = plsc.cumsum(desc) + cum
        hit = (cs >= k_target) & ((cs - desc) < k_target)
        enc = (cs - desc) * 4096 + (bin_base + c * 16 + 15 - lane)
        best = jnp.maximum(best, jnp.max(jnp.where(hit, enc, -1)))
        if nchunks > 1:
            cum = cum + jnp.sum(acc)
    return best & 4095, lax.shift_right_logical(best, 12)


def _sc_topk_body(probsT_hbm, out_hbm, data0_v, data1_v, cidx0_v, cidx1_v,
                  histf0_v, histf1_v, histc0_v, histc1_v, row_v, sem0, sem1):
    wid = lax.axis_index("s") * NC + lax.axis_index("c")
    lane = lax.iota(jnp.int32, 16)
    ones = jnp.ones((16,), jnp.int32)
    zeros16 = jnp.zeros((16,), jnp.int32)

    e0 = wid * EPW
    cp0 = pltpu.async_copy(probsT_hbm.at[e0], data0_v, sem0)
    cp1 = pltpu.async_copy(probsT_hbm.at[e0 + 1], data1_v, sem1)

    @plsc.parallel_loop(0, 8192, 16, unroll=4)
    def _(c):
        histf0_v[pl.ds(c, 16)] = zeros16
        histf1_v[pl.ds(c, 16)] = zeros16

    for c in range(32):
        histc0_v[pl.ds(c * 16, 16)] = zeros16
        histc1_v[pl.ds(c * 16, 16)] = zeros16

    cp0.wait()
    cp1.wait()

    @plsc.parallel_loop(0, CH, 1, unroll=4)
    def _(i):
        par_f = (i & 1) * 4096
        par_c = (i & 1) * 256
        b0 = data0_v[pl.ds(i * 16, 16)]
        b1 = data1_v[pl.ds(i * 16, 16)]
        d0 = lax.shift_right_logical(b0, 19)
        d1 = lax.shift_right_logical(b1, 19)
        plsc.addupdate_scatter(histf0_v, [d0 + par_f], ones)
        plsc.addupdate_scatter(
            histc0_v, [lax.shift_right_logical(d0, 4) + par_c], ones)
        plsc.addupdate_scatter(histf1_v, [d1 + par_f], ones)
        plsc.addupdate_scatter(
            histc1_v, [lax.shift_right_logical(d1, 4) + par_c], ones)

    def resolve_r1(histc_v, histf_v):
        cb, c_above = _scan16(
            lambda c: (histc_v[pl.ds(c * 16, 16)]
                       + histc_v[pl.ds(256 + c * 16, 16)]),
            16, jnp.int32(0), jnp.int32(K), lane)
        fb, f_above = _scan16(
            lambda c, cb=cb: (histf_v[pl.ds(cb * 16, 16)]
                              + histf_v[pl.ds(4096 + cb * 16, 16)]),
            1, c_above, jnp.int32(K), lane, bin_base=0)
        fb = fb + cb * 16
        return fb, K - f_above

    bin0, krem0 = resolve_r1(histc0_v, histf0_v)
    bin1, krem1 = resolve_r1(histc1_v, histf1_v)

    @plsc.parallel_loop(0, CH, 1, unroll=4,
                        carry=(jnp.int32(0), jnp.int32(0)))
    def offs(i, carry):
        o0, o1 = carry
        b0 = data0_v[pl.ds(i * 16, 16)]
        b1 = data1_v[pl.ds(i * 16, 16)]
        idxv = i * 16 + lane
        m0 = lax.shift_right_logical(b0, 19) == bin0
        m1 = lax.shift_right_logical(b1, 19) == bin1
        plsc.store_compressed(cidx0_v.at[pl.ds(o0, 16)], idxv, mask=m0)
        plsc.store_compressed(cidx1_v.at[pl.ds(o1, 16)], idxv, mask=m1)
        return (o0 + plsc.all_reduce_population_count(m0)[0],
                o1 + plsc.all_reduce_population_count(m1)[0])

    l0, l1 = offs

    results = []
    for data_v, cidx_v, histc_v, bin_r1, krem, l_cand in (
            (data0_v, cidx0_v, histc0_v, bin0, krem0, l0),
            (data1_v, cidx1_v, histc1_v, bin1, krem1, l1)):
        nch = lax.shift_right_logical(l_cand + 15, 4)
        prefix = bin_r1
        k_rem = krem

        for sh, w in ((15, 4), (11, 4), (7, 4), (3, 4), (0, 3)):
            histc_v[pl.ds(0, 16)] = zeros16

            def hist_body(i, _, sh=sh, w=w, prefix=prefix, l_cand=l_cand,
                          data_v=data_v, cidx_v=cidx_v, histc_v=histc_v):
                valid = (i * 16 + lane) < l_cand
                ci = cidx_v[pl.ds(i * 16, 16)] & (N - 1)
                b = plsc.load_gather(data_v, [ci], mask=valid)
                m = valid & (lax.shift_right_logical(b, sh + w) == prefix)
                dig = lax.shift_right_logical(b, sh) & ((1 << w) - 1)
                plsc.addupdate_scatter(histc_v, [dig], ones, mask=m)
                return 0

            lax.fori_loop(0, nch, hist_body, 0)

            bv, cbv = _scan16(
                lambda c, histc_v=histc_v: histc_v[pl.ds(0, 16)],
                1, jnp.int32(0), k_rem, lane)
            prefix = prefix * (1 << w) + bv
            k_rem = k_rem - cbv

        t_bits = prefix

        def tie_body(i, carry, t_bits=t_bits, k_rem=k_rem, l_cand=l_cand,
                     data_v=data_v, cidx_v=cidx_v):
            run, p = carry
            valid = (i * 16 + lane) < l_cand
            ci = cidx_v[pl.ds(i * 16, 16)] & (N - 1)
            b = plsc.load_gather(data_v, [ci], mask=valid)
            tie = valid & (b == t_bits)
            ti = tie.astype(jnp.int32)
            cs = plsc.cumsum(ti)
            hit = tie & ((run + cs) == k_rem)
            p = jnp.maximum(p, jnp.max(jnp.where(hit, ci, -1)))
            return (run + jnp.sum(ti), p)

        _, p_cut = lax.fori_loop(0, nch, tie_body,
                                 (jnp.int32(0), jnp.int32(-1)))
        results.append((t_bits, p_cut))

    (t0, p0), (t1, p1) = results
    row = jnp.where(lane == 0, t0,
          jnp.where(lane == 1, t1,
          jnp.where(lane == 2, p0,
          jnp.where(lane == 3, p1, jnp.int32(0)))))
    row_v[...] = row
    pltpu.sync_copy(row_v, out_hbm.at[wid])


_sc_topk = functools.partial(
    pl.kernel,
    out_type=jax.ShapeDtypeStruct((NW, 16), jnp.int32),
    mesh=plsc.VectorSubcoreMesh(core_axis_name="c", subcore_axis_name="s"),
    compiler_params=pltpu.CompilerParams(needs_layout_passes=False),
    scratch_types=[
        pltpu.VMEM((N,), jnp.int32),
        pltpu.VMEM((N,), jnp.int32),
        pltpu.VMEM((N,), jnp.int32),
        pltpu.VMEM((N,), jnp.int32),
        pltpu.VMEM((8192,), jnp.int32),
        pltpu.VMEM((8192,), jnp.int32),
        pltpu.VMEM((512,), jnp.int32),
        pltpu.VMEM((512,), jnp.int32),
        pltpu.VMEM((16,), jnp.int32),
        pltpu.SemaphoreType.DMA,
        pltpu.SemaphoreType.DMA,
    ],
)(_sc_topk_body)


def _mask_body(probs_ref, t_ref, cut_ref, disp_ref, comb_ref):
    p = probs_ref[...]
    bits = lax.bitcast_convert_type(p, jnp.int32)
    t = t_ref[...]
    cut = cut_ref[...]
    idx = lax.broadcasted_iota(jnp.int32, (N, E), 0)
    sel = (bits > t) | ((bits == t) & (idx <= cut))
    disp = jnp.where(sel, p, 0.0)
    dsum = jnp.sum(disp, axis=1, keepdims=True)
    comb = jnp.where(dsum > 0, disp / dsum, 0.0)
    disp_ref[...] = disp
    comb_ref[...] = comb


@jax.jit
def kernel(hidden_states, W):
    x = hidden_states.reshape(N, H)
    probs, probsT = pl.pallas_call(
        _router_body,
        grid=(GRID_A,),
        in_specs=[
            pl.BlockSpec((TILE, H), lambda i: (i, 0)),
            pl.BlockSpec((E, H), lambda i: (0, 0)),
        ],
        out_specs=[
            pl.BlockSpec((TILE, E), lambda i: (i, 0)),
            pl.BlockSpec((E, TILE), lambda i: (0, i)),
        ],
        out_shape=[
            jax.ShapeDtypeStruct((N, E), jnp.float32),
            jax.ShapeDtypeStruct((E, N), jnp.int32),
        ],
    )(x, W)

    sc_out = _sc_topk(probsT)
    t_bits = sc_out[:, 0:2].reshape(1, E)
    cut = sc_out[:, 2:4].reshape(1, E)

    disp, comb = pl.pallas_call(
        _mask_body,
        out_shape=[
            jax.ShapeDtypeStruct((N, E), jnp.float32),
            jax.ShapeDtypeStruct((N, E), jnp.float32),
        ],
    )(probs, t_bits, cut)

    shape = (B, S, E)
    return (disp.reshape(shape), comb.reshape(shape),
            jnp.array(0.0, dtype=jnp.float32), probs.reshape(shape))


# speedup vs baseline: 1.0010x; 1.0010x over previous
import functools

import jax
import jax.numpy as jnp
from jax import lax
from jax.experimental import pallas as pl
from jax.experimental.pallas import tpu as pltpu
from jax.experimental.pallas import tpu_sc as plsc

B, S, H, E = 4, 2048, 4096, 64
N = B * S
K = 160

TILE = 512
GRID_A = N // TILE

NC, NS = 2, 16
NW = NC * NS
EPW = E // NW
CH = N // 16


def _router_body(x_ref, w_ref, probs_ref, probsT_ref):
    x = x_ref[...]
    w = w_ref[...]
    logits = lax.dot_general(
        x, w, (((1,), (1,)), ((), ())),
        preferred_element_type=jnp.float32,
        precision=lax.Precision.DEFAULT,
    )
    m = jnp.max(logits, axis=1, keepdims=True)
    e = jnp.exp(logits - m)
    probs = e / jnp.sum(e, axis=1, keepdims=True)
    probs_ref[...] = probs
    probsT_ref[...] = lax.bitcast_convert_type(probs.T, jnp.int32)


def _scan16(chunk_loader, nchunks, base_count, k_target, lane, bin_base=0):
    cum = base_count
    best = jnp.int32(-1)
    for c in range(nchunks - 1, -1, -1):
        acc = chunk_loader(c)
        desc = lax.rev(acc, (0,))
        cs = plsc.cumsum(desc) + cum
        hit = (cs >= k_target) & ((cs - desc) < k_target)
        enc = (cs - desc) * 4096 + (bin_base + c * 16 + 15 - lane)
        best = jnp.maximum(best, jnp.max(jnp.where(hit, enc, -1)))
        if nchunks > 1:
            cum = cum + jnp.sum(acc)
    return best & 4095, lax.shift_right_logical(best, 12)


def _sc_topk_body(probsT_hbm, out_hbm, data0_v, data1_v, cidx0_v, cidx1_v,
                  histf0_v, histf1_v, histc0_v, histc1_v, row_v, sem0, sem1):
    wid = lax.axis_index("s") * NC + lax.axis_index("c")
    lane = lax.iota(jnp.int32, 16)
    ones = jnp.ones((16,), jnp.int32)
    zeros16 = jnp.zeros((16,), jnp.int32)

    e0 = wid * EPW
    cp0 = pltpu.async_copy(probsT_hbm.at[e0], data0_v, sem0)
    cp1 = pltpu.async_copy(probsT_hbm.at[e0 + 1], data1_v, sem1)

    @plsc.parallel_loop(0, 4096, 16, unroll=4)
    def _(c):
        histf0_v[pl.ds(c, 16)] = zeros16
        histf1_v[pl.ds(c, 16)] = zeros16

    for c in range(16):
        histc0_v[pl.ds(c * 16, 16)] = zeros16
        histc1_v[pl.ds(c * 16, 16)] = zeros16

    cp0.wait()
    cp1.wait()

    @plsc.parallel_loop(0, CH, 1, unroll=4)
    def _(i):
        b0 = data0_v[pl.ds(i * 16, 16)]
        b1 = data1_v[pl.ds(i * 16, 16)]
        d0 = lax.shift_right_logical(b0, 19)
        d1 = lax.shift_right_logical(b1, 19)
        plsc.addupdate_scatter(histf0_v, [d0], ones)
        plsc.addupdate_scatter(histc0_v, [lax.shift_right_logical(d0, 4)],
                               ones)
        plsc.addupdate_scatter(histf1_v, [d1], ones)
        plsc.addupdate_scatter(histc1_v, [lax.shift_right_logical(d1, 4)],
                               ones)

    def resolve_r1(histc_v, histf_v):
        cb, c_above = _scan16(
            lambda c: histc_v[pl.ds(c * 16, 16)], 16, jnp.int32(0),
            jnp.int32(K), lane)
        fb, f_above = _scan16(
            lambda c, cb=cb: histf_v[pl.ds(cb * 16, 16)], 1, c_above,
            jnp.int32(K), lane, bin_base=0)
        fb = fb + cb * 16
        return fb, K - f_above

    bin0, krem0 = resolve_r1(histc0_v, histf0_v)
    bin1, krem1 = resolve_r1(histc1_v, histf1_v)

    @plsc.parallel_loop(0, CH, 1, unroll=4,
                        carry=(jnp.int32(0), jnp.int32(0)))
    def offs(i, carry):
        o0, o1 = carry
        b0 = data0_v[pl.ds(i * 16, 16)]
        b1 = data1_v[pl.ds(i * 16, 16)]
        idxv = i * 16 + lane
        m0 = lax.shift_right_logical(b0, 19) == bin0
        m1 = lax.shift_right_logical(b1, 19) == bin1
        plsc.store_compressed(cidx0_v.at[pl.ds(o0, 16)], idxv, mask=m0)
        plsc.store_compressed(cidx1_v.at[pl.ds(o1, 16)], idxv, mask=m1)
        return (o0 + plsc.all_reduce_population_count(m0)[0],
                o1 + plsc.all_reduce_population_count(m1)[0])

    l0, l1 = offs

    results = []
    for data_v, cidx_v, histc_v, bin_r1, krem, l_cand in (
            (data0_v, cidx0_v, histc0_v, bin0, krem0, l0),
            (data1_v, cidx1_v, histc1_v, bin1, krem1, l1)):
        nch = lax.shift_right_logical(l_cand + 15, 4)
        prefix = bin_r1
        k_rem = krem

        for sh, w in ((15, 4), (11, 4), (7, 4), (3, 4), (0, 3)):
            histc_v[pl.ds(0, 16)] = zeros16

            def hist_body(i, _, sh=sh, w=w, prefix=prefix, l_cand=l_cand,
                          data_v=data_v, cidx_v=cidx_v, histc_v=histc_v):
                valid = (i * 16 + lane) < l_cand
                ci = cidx_v[pl.ds(i * 16, 16)] & (N - 1)
                b = plsc.load_gather(data_v, [ci], mask=valid)
                m = valid & (lax.shift_right_logical(b, sh + w) == prefix)
                dig = lax.shift_right_logical(b, sh) & ((1 << w) - 1)
                plsc.addupdate_scatter(histc_v, [dig], ones, mask=m)
                return 0

            lax.fori_loop(0, nch, hist_body, 0)

            bv, cbv = _scan16(
                lambda c, histc_v=histc_v: histc_v[pl.ds(0, 16)],
                1, jnp.int32(0), k_rem, lane)
            prefix = prefix * (1 << w) + bv
            k_rem = k_rem - cbv

        t_bits = prefix

        def tie_body(i, carry, t_bits=t_bits, k_rem=k_rem, l_cand=l_cand,
                     data_v=data_v, cidx_v=cidx_v):
            run, p = carry
            valid = (i * 16 + lane) < l_cand
            ci = cidx_v[pl.ds(i * 16, 16)] & (N - 1)
            b = plsc.load_gather(data_v, [ci], mask=valid)
            tie = valid & (b == t_bits)
            ti = tie.astype(jnp.int32)
            cs = plsc.cumsum(ti)
            hit = tie & ((run + cs) == k_rem)
            p = jnp.maximum(p, jnp.max(jnp.where(hit, ci, -1)))
            return (run + jnp.sum(ti), p)

        _, p_cut = lax.fori_loop(0, nch, tie_body,
                                 (jnp.int32(0), jnp.int32(-1)))
        results.append((t_bits, p_cut))

    (t0, p0), (t1, p1) = results
    row = jnp.where(lane == 0, t0,
          jnp.where(lane == 1, t1,
          jnp.where(lane == 2, p0,
          jnp.where(lane == 3, p1, jnp.int32(0)))))
    row_v[...] = row
    pltpu.sync_copy(row_v, out_hbm.at[wid])


_sc_topk = functools.partial(
    pl.kernel,
    out_type=jax.ShapeDtypeStruct((NW, 16), jnp.int32),
    mesh=plsc.VectorSubcoreMesh(core_axis_name="c", subcore_axis_name="s"),
    compiler_params=pltpu.CompilerParams(needs_layout_passes=False),
    scratch_types=[
        pltpu.VMEM((N,), jnp.int32),
        pltpu.VMEM((N,), jnp.int32),
        pltpu.VMEM((N,), jnp.int32),
        pltpu.VMEM((N,), jnp.int32),
        pltpu.VMEM((4096,), jnp.int32),
        pltpu.VMEM((4096,), jnp.int32),
        pltpu.VMEM((256,), jnp.int32),
        pltpu.VMEM((256,), jnp.int32),
        pltpu.VMEM((16,), jnp.int32),
        pltpu.SemaphoreType.DMA,
        pltpu.SemaphoreType.DMA,
    ],
)(_sc_topk_body)


def _mask_body(probs_ref, t_ref, cut_ref, disp_ref, comb_ref):
    p = probs_ref[...]
    bits = lax.bitcast_convert_type(p, jnp.int32)
    t = t_ref[...]
    cut = cut_ref[...]
    idx = lax.broadcasted_iota(jnp.int32, (N, E), 0)
    sel = (bits > t) | ((bits == t) & (idx <= cut))
    disp = jnp.where(sel, p, 0.0)
    dsum = jnp.sum(disp, axis=1, keepdims=True)
    comb = jnp.where(dsum > 0, disp / dsum, 0.0)
    disp_ref[...] = disp
    comb_ref[...] = comb


@jax.jit
def kernel(hidden_states, W):
    x = hidden_states.reshape(N, H)
    probs, probsT = pl.pallas_call(
        _router_body,
        grid=(GRID_A,),
        in_specs=[
            pl.BlockSpec((TILE, H), lambda i: (i, 0)),
            pl.BlockSpec((E, H), lambda i: (0, 0)),
        ],
        out_specs=[
            pl.BlockSpec((TILE, E), lambda i: (i, 0)),
            pl.BlockSpec((E, TILE), lambda i: (0, i)),
        ],
        out_shape=[
            jax.ShapeDtypeStruct((N, E), jnp.float32),
            jax.ShapeDtypeStruct((E, N), jnp.int32),
        ],
    )(x, W)

    sc_out = _sc_topk(probsT)
    t_bits = sc_out[:, 0:2].reshape(1, E)
    cut = sc_out[:, 2:4].reshape(1, E)

    disp, comb = pl.pallas_call(
        _mask_body,
        out_shape=[
            jax.ShapeDtypeStruct((N, E), jnp.float32),
            jax.ShapeDtypeStruct((N, E), jnp.float32),
        ],
    )(probs, t_bits, cut)

    shape = (B, S, E)
    return (disp.reshape(shape), comb.reshape(shape),
            jnp.array(0.0, dtype=jnp.float32), probs.reshape(shape))
